# Initial kernel scaffold; baseline (speedup 1.0000x reference)
import jax
import jax.numpy as jnp
from jax.experimental import pallas as pl


def kernel(targets, types, neighbors, base_node_embeddings, node_type_embeddings, trans_weights, trans_weights_s1, trans_weights_s2):
    raise NotImplementedError("write your pallas kernel here")


# speedup vs baseline: 3.6096x; 3.6096x over previous
import functools

import jax
import jax.numpy as jnp
from jax import lax
from jax.experimental import pallas as pl
from jax.experimental.pallas import tpu as pltpu
from jax.experimental.pallas import tpu_sc as plsc

B = 4096
V = 1000000
ET = 2
EMB = 64
EEMB = 16
ATT = 32
NS = 20

_L = 16
_NC = 2
_NSUB = 16
_NW = _NC * _NSUB
_BPW = B // _NW
_PAIRS = _BPW * ET
_NIDX = _PAIRS * NS
_CHUNK = 128
_NCHUNK = _NIDX // _CHUNK
_VPC = _CHUNK // _L


def _sc_body(nbr_hbm, tgt_hbm, table_hbm, base_hbm, agg_out, ne_out,
             nbr_v, idx_v, rows_v, agg_v, tgt_v, brows_v, sem_b, sem_g):
    wid = lax.axis_index("s") * _NC + lax.axis_index("c")
    nbase = wid * _NIDX

    pltpu.sync_copy(nbr_hbm.at[pl.ds(nbase, _NIDX)], nbr_v)
    pltpu.sync_copy(tgt_hbm.at[pl.ds(wid * _BPW, _BPW)], tgt_v)
    base_cp = pltpu.async_copy(base_hbm.at[tgt_v], brows_v, sem_b)

    def _sr(sr, _):
        lane = lax.iota(jnp.int32, _L)
        step4 = (lane + 12) >> 4
        step8 = (lane + 8) >> 4
        step12 = (lane + 4) >> 4
        et_pat = [
            None,
            step4,
            1 - step8,
            step12,
            1,
        ]
        for v in range(40):
            src = nbr_v[pl.ds(sr * (40 * _L) + v * _L, _L)]
            pat = et_pat[v % 5]
            idx = src * ET if pat is None else src * ET + pat
            idx_v[sr * 5 + (v // 8), pl.ds((v % 8) * _L, _L)] = idx
        return 0
    lax.fori_loop(0, 8, _sr, 0)

    copies = [pltpu.async_copy(table_hbm.at[idx_v.at[r]],
                               rows_v.at[pl.ds(r * _CHUNK, _CHUNK)], sem_g)
              for r in range(_NCHUNK)]
    for cp in copies:
        cp.wait()

    def _red(j, _):
        acc = rows_v[j * NS]
        for s in range(1, NS):
            acc = acc + rows_v[j * NS + s]
        agg_v[j] = acc
        return 0
    lax.fori_loop(0, _PAIRS, _red, 0)

    pltpu.sync_copy(agg_v, agg_out.at[pl.ds(wid * _PAIRS, _PAIRS)])
    base_cp.wait()
    pltpu.sync_copy(brows_v, ne_out.at[pl.ds(wid * _BPW, _BPW)])


@functools.cache
def _make_sc_gather():
    return functools.partial(
        pl.kernel,
        out_type=[jax.ShapeDtypeStruct((B * ET, EEMB), jnp.float32),
                  jax.ShapeDtypeStruct((B, EMB), jnp.float32)],
        mesh=plsc.VectorSubcoreMesh(core_axis_name="c", subcore_axis_name="s"),
        compiler_params=pltpu.CompilerParams(use_tc_tiling_on_sc=False),
        scratch_types=[
            pltpu.VMEM((_NIDX,), jnp.int32),
            pltpu.VMEM((_NCHUNK, _CHUNK), jnp.int32),
            pltpu.VMEM((_NIDX, EEMB), jnp.float32),
            pltpu.VMEM((_PAIRS, EEMB), jnp.float32),
            pltpu.VMEM((_BPW,), jnp.int32),
            pltpu.VMEM((_BPW, EMB), jnp.float32),
            pltpu.SemaphoreType.DMA,
            pltpu.SemaphoreType.DMA,
        ],
    )(_sc_body)


def _tc_body(agg_ref, ne_ref, t_ref, s10_ref, s11_ref, s20_ref, s21_ref,
             w0_ref, w1_ref, o_ref):
    a = agg_ref[...] * (1.0 / NS)
    a0 = a[:, :EEMB]
    a1 = a[:, EEMB:]
    t = t_ref[...]
    tn = 1.0 - t

    def _logit(ai):
        h0 = jnp.tanh(jnp.dot(ai, s10_ref[...],
                              preferred_element_type=jnp.float32,
                              precision=lax.Precision.HIGHEST))
        h1 = jnp.tanh(jnp.dot(ai, s11_ref[...],
                              preferred_element_type=jnp.float32,
                              precision=lax.Precision.HIGHEST))
        l0 = jnp.sum(h0 * s20_ref[...], axis=1, keepdims=True)
        l1 = jnp.sum(h1 * s21_ref[...], axis=1, keepdims=True)
        return l0 * tn + l1 * t

    la = _logit(a0)
    lb = _logit(a1)
    m = jnp.maximum(la, lb)
    ea = jnp.exp(la - m)
    eb = jnp.exp(lb - m)
    inv = 1.0 / (ea + eb)
    na = (ea * inv) * a0 + (eb * inv) * a1
    o0 = jnp.dot(na, w0_ref[...], preferred_element_type=jnp.float32,
                 precision=lax.Precision.HIGHEST)
    o1 = jnp.dot(na, w1_ref[...], preferred_element_type=jnp.float32,
                 precision=lax.Precision.HIGHEST)
    allv = ne_ref[...] + o0 * tn + o1 * t
    sq = jnp.sum(allv * allv, axis=1, keepdims=True)
    o_ref[...] = allv * lax.rsqrt(jnp.maximum(sq, 1e-12))


_TC_BLK = 512
_TC_GRID = B // _TC_BLK


def _tc_combine(agg2, ne, tf, s10, s11, s20, s21, w0, w1):
    fixed = lambda i: (0, 0)
    row = lambda i: (i, 0)
    return pl.pallas_call(
        _tc_body,
        grid=(_TC_GRID,),
        in_specs=[
            pl.BlockSpec((_TC_BLK, ET * EEMB), row),
            pl.BlockSpec((_TC_BLK, EMB), row),
            pl.BlockSpec((_TC_BLK, 1), row),
            pl.BlockSpec((EEMB, ATT), fixed),
            pl.BlockSpec((EEMB, ATT), fixed),
            pl.BlockSpec((1, ATT), fixed),
            pl.BlockSpec((1, ATT), fixed),
            pl.BlockSpec((EEMB, EMB), fixed),
            pl.BlockSpec((EEMB, EMB), fixed),
        ],
        out_specs=pl.BlockSpec((_TC_BLK, EMB), row),
        out_shape=jax.ShapeDtypeStruct((B, EMB), jnp.float32),
    )(agg2, ne, tf, s10, s11, s20, s21, w0, w1)


def kernel(targets, types, neighbors, base_node_embeddings,
           node_type_embeddings, trans_weights, trans_weights_s1,
           trans_weights_s2):
    nbr = neighbors.reshape(-1).astype(jnp.int32)
    tgt = targets.astype(jnp.int32)
    table = node_type_embeddings.reshape(V * ET, EEMB)

    agg, ne = _make_sc_gather()(nbr, tgt, table, base_node_embeddings)

    agg2 = agg.reshape(B, ET * EEMB)
    tf = types.astype(jnp.float32).reshape(B, 1)
    return _tc_combine(
        agg2, ne, tf,
        trans_weights_s1[0], trans_weights_s1[1],
        trans_weights_s2[0].reshape(1, ATT), trans_weights_s2[1].reshape(1, ATT),
        trans_weights[0], trans_weights[1])
